# baseline (speedup 1.0000x reference)
import functools

import jax
import jax.numpy as jnp
import numpy as np
from jax import lax
from jax.experimental import pallas as pl
from jax.experimental.pallas import tpu as pltpu
from jax.experimental.pallas import tpu_sc as plsc

F32 = jnp.float32
NC, NS, LANES = 2, 16, 16
NW = NC * NS
EC = 80
W = 128

_MESH = plsc.VectorSubcoreMesh(
    core_axis_name="c", subcore_axis_name="s", num_cores=NC, num_subcores=NS
)

_GDN = lax.GatherDimensionNumbers(
    offset_dims=(), collapsed_slice_dims=(0,), start_index_map=(0,)
)


def _vgather(v, idx):
    return lax.gather(v, idx[:, None], _GDN, (1,),
                      mode=lax.GatherScatterMode.PROMISE_IN_BOUNDS)


def _prep1_body(x_ref, w_ref, msa_ref, mtb_ref, mhb_ref, a_ref, b_ref):
    h = jnp.dot(x_ref[...], w_ref[...], preferred_element_type=F32)
    lr = jnp.where(h > 0, h, 0.2 * h)
    a_ref[...] = jnp.dot(lr, msa_ref[...], preferred_element_type=F32)
    b_ref[...] = (jnp.dot(lr, mtb_ref[...], preferred_element_type=F32)
                  + jnp.dot(h, mhb_ref[...], preferred_element_type=F32))


def _prep2_body(p0_ref, p1_ref, md_ref, mn_ref, w2_ref, msa2_ref, mtb2_ref,
                mhb2_ref, a2_ref, b2_ref):
    acc = p0_ref[...] + p1_ref[...]
    den = jnp.dot(acc, md_ref[...], preferred_element_type=F32)
    num = jnp.dot(acc, mn_ref[...], preferred_element_type=F32)
    x2 = jnp.maximum(num / (den + 1e-9), 0.0)
    h2 = jnp.dot(x2, w2_ref[...], preferred_element_type=F32)
    lr2 = jnp.where(h2 > 0, h2, 0.2 * h2)
    a2_ref[...] = jnp.dot(lr2, msa2_ref[...], preferred_element_type=F32)
    b2_ref[...] = (jnp.dot(lr2, mtb2_ref[...], preferred_element_type=F32)
                   + jnp.dot(h2, mhb2_ref[...], preferred_element_type=F32))


ECP = 64
IDXR = 4
BPC = IDXR * W // ECP


def _edge_body(src2_hbm, tgt2_hbm, a_hbm, b_hbm, out_hbm, acc_sh,
               i2s0, i2s1, i2t0, i2t1, s0, s1, t0, t1,
               sb0, sb1, tb0, tb1, contrib,
               sem_a0, sem_a1, sem_b0, sem_b1, sem_i0, sem_i1, sem_s,
               *, layer):
    core = lax.axis_index("c")
    sub = lax.axis_index("s")
    wid = sub * NC + core
    n_pad = acc_sh.shape[0]
    rows_per = src2_hbm.shape[0] // NW
    nchunks_i = rows_per // IDXR
    npairs = nchunks_i // 2
    rps = n_pad // NS
    nchunks_w = rps // ECP
    zeros16 = jnp.zeros((LANES,), F32)
    ii = lax.iota(jnp.int32, LANES)
    rs = sub * rps
    rbase = wid * rows_per

    def _zrow(i, c):
        for o in range(0, W, LANES):
            contrib[i, pl.ds(o, LANES)] = zeros16
        return c
    lax.fori_loop(0, ECP, _zrow, 0)
    for k in range(nchunks_w):
        pltpu.sync_copy(contrib, acc_sh.at[pl.ds(rs + k * ECP, ECP)])
    plsc.subcore_barrier()

    bidx = [(ii >> 3) + 2 * c for c in range(4)]
    zidx = ii * 0
    ibufs = [(i2s0, i2t0, sem_i0), (i2s1, i2t1, sem_i1)]
    sets = [(s0, t0, sb0, tb0, sem_a0, sem_b0), (s1, t1, sb1, tb1, sem_a1, sem_b1)]

    def _issue_idx(g, p):
        isb, itb, sem = ibufs[p]
        pltpu.async_copy(src2_hbm.at[pl.ds(rbase + g * IDXR, IDXR)], isb, sem)
        pltpu.async_copy(tgt2_hbm.at[pl.ds(rbase + g * IDXR, IDXR)], itb, sem)

    def _wait_idx(p):
        isb, itb, sem = ibufs[p]
        pltpu.make_async_copy(src2_hbm.at[pl.ds(0, IDXR)], isb, sem).wait()
        pltpu.make_async_copy(tgt2_hbm.at[pl.ds(0, IDXR)], itb, sem).wait()

    def _extract(j, p, ip):
        isb, itb, _ = ibufs[ip]
        sv, tv = sets[p][0], sets[p][1]
        r = j >> 1
        cb = (j & 1) * ECP
        for o in range(0, ECP, LANES):
            sv[pl.ds(o, LANES)] = isb[r, pl.ds(cb + o, LANES)]
            tv[pl.ds(o, LANES)] = itb[r, pl.ds(cb + o, LANES)]

    def _issue_gather(p):
        sv, tv, sb, tb, sa, sb_sem = sets[p]
        pltpu.async_copy(a_hbm.at[sv], sb, sa)
        pltpu.async_copy(b_hbm.at[tv], tb, sb_sem)

    def _wait_gather(p):
        sv, tv, sb, tb, sa, sb_sem = sets[p]
        pltpu.make_async_copy(a_hbm.at[sv], sb, sa).wait()
        pltpu.make_async_copy(b_hbm.at[tv], tb, sb_sem).wait()

    def _wait_scatter():
        pltpu.make_async_copy(contrib, acc_sh.at[s0], sem_s).wait()

    def _issue_scatter(p):
        pltpu.async_copy(contrib, acc_sh.at[sets[p][0]], sem_s, add=True)

    if layer == 1:
        def _mk_edge(sb, tb):
            def _edge(i, c2):
                sv = sb[i, pl.ds(0, LANES)]
                tv = tb[i, pl.ds(0, LANES)]
                ev = jnp.exp(jnp.clip(sv + tv, -2.0, 2.0))
                contrib[i, pl.ds(0, LANES)] = ev
                for v in range(4):
                    hc = tb[i, pl.ds(8 + LANES * v, LANES)]
                    eb = _vgather(ev, bidx[v])
                    contrib[i, pl.ds(8 + LANES * v, LANES)] = eb * hc
                return c2
            return _edge
    else:
        def _mk_edge(sb, tb):
            def _edge(i, c2):
                sv = sb[i, pl.ds(0, LANES)]
                tv = tb[i, pl.ds(0, LANES)]
                sm = sv + tv
                ev = jnp.exp(jnp.clip(sm, -2.0, 2.0))
                e2 = _vgather(ev, zidx)
                contrib[i, pl.ds(0, LANES)] = e2 * jnp.where(ii == 0, 1.0, sm)
                return c2
            return _edge

    def _compute(p):
        lax.fori_loop(0, ECP, _mk_edge(sets[p][2], sets[p][3]), 0)

    pltpu.sync_copy(src2_hbm.at[pl.ds(rbase, IDXR)], i2s0)
    pltpu.sync_copy(tgt2_hbm.at[pl.ds(rbase, IDXR)], i2t0)
    _extract(0, 0, 0)
    _issue_gather(0)
    _issue_idx(1, 1)

    def _pair(g2, c):
        for jj in range(2 * BPC):
            p = jj & 1
            ip = jj // BPC
            _wait_gather(p)
            if jj == 0:
                @pl.when(g2 > 0)
                def _(_p=p):
                    _wait_scatter()
            else:
                _wait_scatter()
            j_in_chunk = jj % BPC
            if j_in_chunk < BPC - 1:
                _extract(j_in_chunk + 1, p ^ 1, ip)
                _issue_gather(p ^ 1)
            elif jj == BPC - 1:
                _wait_idx(1)
                _extract(0, p ^ 1, 1)
                _issue_gather(p ^ 1)

                @pl.when(g2 < npairs - 1)
                def _():
                    _issue_idx(2 * g2 + 2, 0)
            else:
                @pl.when(g2 < npairs - 1)
                def _(_p=p):
                    _wait_idx(0)
                    _extract(0, _p ^ 1, 0)
                    _issue_gather(_p ^ 1)
                    _issue_idx(2 * g2 + 3, 1)
            _compute(p)
            _issue_scatter(p)
        return c
    lax.fori_loop(0, npairs, _pair, 0)
    _wait_scatter()

    plsc.subcore_barrier()
    for k in range(nchunks_w):
        pltpu.sync_copy(acc_sh.at[pl.ds(rs + k * ECP, ECP)], contrib)
        pltpu.sync_copy(contrib, out_hbm.at[core, pl.ds(rs + k * ECP, ECP)])


def _fin_body(q0_hbm, q1_hbm, idx_hbm, out_hbm, idxv, r0, r1, sem0, sem1):
    core = lax.axis_index("c")
    sub = lax.axis_index("s")
    wid = sub * NC + core
    per = idx_hbm.shape[0] // NW
    nb = per // EC
    ii = lax.iota(jnp.int32, LANES)
    zidx = ii * 0

    def _block(b, c):
        base = wid * per + b * EC
        pltpu.sync_copy(idx_hbm.at[pl.ds(base, EC)], idxv)
        cp0 = pltpu.async_copy(q0_hbm.at[idxv], r0, sem0)
        cp1 = pltpu.async_copy(q1_hbm.at[idxv], r1, sem1)
        cp0.wait()
        cp1.wait()

        def _row(i, c2):
            srow = r0[i, pl.ds(0, LANES)] + r1[i, pl.ds(0, LANES)]
            den = _vgather(srow, zidx)
            r0[i, pl.ds(0, LANES)] = jnp.maximum(srow / (den + 1e-9), 0.0)
            return c2
        lax.fori_loop(0, EC, _row, 0)
        pltpu.sync_copy(r0, out_hbm.at[pl.ds(base, EC)])
        return c
    lax.fori_loop(0, nb, _block, 0)


def _make_edge_call(n_pad, layer):
    body = functools.partial(_edge_body, layer=layer)
    return pl.kernel(
        body,
        out_type=jax.ShapeDtypeStruct((NC, n_pad, W), F32),
        mesh=_MESH,
        scratch_types=[
            pltpu.VMEM_SHARED((n_pad, W), F32),
            pltpu.VMEM((IDXR, W), jnp.int32),
            pltpu.VMEM((IDXR, W), jnp.int32),
            pltpu.VMEM((IDXR, W), jnp.int32),
            pltpu.VMEM((IDXR, W), jnp.int32),
            pltpu.VMEM((ECP,), jnp.int32),
            pltpu.VMEM((ECP,), jnp.int32),
            pltpu.VMEM((ECP,), jnp.int32),
            pltpu.VMEM((ECP,), jnp.int32),
            pltpu.VMEM((ECP, W), F32),
            pltpu.VMEM((ECP, W), F32),
            pltpu.VMEM((ECP, W), F32),
            pltpu.VMEM((ECP, W), F32),
            pltpu.VMEM((ECP, W), F32),
            pltpu.SemaphoreType.DMA,
            pltpu.SemaphoreType.DMA,
            pltpu.SemaphoreType.DMA,
            pltpu.SemaphoreType.DMA,
            pltpu.SemaphoreType.DMA,
            pltpu.SemaphoreType.DMA,
            pltpu.SemaphoreType.DMA,
        ],
    )


def kernel(node_states, edges, indices, W1, a1, W2, a2):
    n, d = node_states.shape
    h1, _, u1 = W1.shape
    hu = h1 * u1
    od = W2.shape[2]
    nidx = indices.shape[0]

    w1f = jnp.transpose(W1, (1, 0, 2)).reshape(d, hu)
    rows = np.arange(hu)
    heads = rows // u1
    msa = jnp.zeros((hu, W), F32).at[rows, heads].set(a1[:, :u1, 0].reshape(hu))
    mtb = jnp.zeros((hu, W), F32).at[rows, heads].set(a1[:, u1:, 0].reshape(hu))
    mhb = np.zeros((hu, W), np.float32)
    mhb[rows, rows + 8] = 1.0
    mhb = jnp.asarray(mhb)
    md = np.zeros((W, hu), np.float32)
    md[heads, rows] = 1.0
    md = jnp.asarray(md)
    mn = np.zeros((W, hu), np.float32)
    mn[rows + 8, rows] = 1.0
    mn = jnp.asarray(mn)
    w2p = jnp.concatenate([W2[0], jnp.zeros((hu, 8 - od), F32)], axis=1)
    j7 = np.arange(od)
    msa2 = jnp.zeros((8, W), F32).at[j7, 0].set(a2[0, :od, 0])
    mtb2 = jnp.zeros((8, W), F32).at[j7, 0].set(a2[0, od:, 0])
    mhb2 = np.zeros((8, W), np.float32)
    mhb2[j7, j7 + 1] = 1.0
    mhb2 = jnp.asarray(mhb2)

    e_num = edges.shape[0]
    per_tile = ((e_num + NW * IDXR * W - 1) // (NW * IDXR * W)) * IDXR * W
    e_pad = NW * per_tile
    src = jnp.concatenate(
        [edges[:, 0], jnp.full((e_pad - e_num,), n, jnp.int32)]).reshape(-1, W)
    tgt = jnp.concatenate(
        [edges[:, 1], jnp.zeros((e_pad - e_num,), jnp.int32)]).reshape(-1, W)

    npd = ((n + NS * EC - 1) // (NS * EC)) * (NS * EC)
    xp = jnp.concatenate([node_states, jnp.zeros((npd - n, d), F32)], axis=0)
    blk = 2048
    grid = (npd // blk,)
    full = lambda i: (0, 0)
    rowb = lambda i: (i, 0)
    prep1 = pl.pallas_call(
        _prep1_body,
        grid=grid,
        in_specs=[
            pl.BlockSpec((blk, d), rowb),
            pl.BlockSpec((d, hu), full),
            pl.BlockSpec((hu, W), full),
            pl.BlockSpec((hu, W), full),
            pl.BlockSpec((hu, W), full),
        ],
        out_specs=[pl.BlockSpec((blk, W), rowb), pl.BlockSpec((blk, W), rowb)],
        out_shape=[jax.ShapeDtypeStruct((npd, W), F32),
                   jax.ShapeDtypeStruct((npd, W), F32)],
    )
    tab_a1, tab_b1 = prep1(xp, w1f, msa, mtb, mhb)

    edge1 = _make_edge_call(npd, 1)
    p1 = edge1(src, tgt, tab_a1, tab_b1)

    prep2 = pl.pallas_call(
        _prep2_body,
        grid=grid,
        in_specs=[
            pl.BlockSpec((blk, W), rowb),
            pl.BlockSpec((blk, W), rowb),
            pl.BlockSpec((W, hu), full),
            pl.BlockSpec((W, hu), full),
            pl.BlockSpec((hu, 8), full),
            pl.BlockSpec((8, W), full),
            pl.BlockSpec((8, W), full),
            pl.BlockSpec((8, W), full),
        ],
        out_specs=[pl.BlockSpec((blk, W), rowb), pl.BlockSpec((blk, W), rowb)],
        out_shape=[jax.ShapeDtypeStruct((npd, W), F32),
                   jax.ShapeDtypeStruct((npd, W), F32)],
    )
    tab_a2, tab_b2 = prep2(p1[0], p1[1], md, mn, w2p, msa2, mtb2, mhb2)

    edge2 = _make_edge_call(npd, 2)
    q = edge2(src, tgt, tab_a2, tab_b2)

    npad = ((nidx + NW * EC - 1) // (NW * EC)) * (NW * EC)
    idxp = jnp.concatenate([indices, jnp.zeros((npad - nidx,), jnp.int32)])
    fin = pl.kernel(
        _fin_body,
        out_type=jax.ShapeDtypeStruct((npad, W), F32),
        mesh=_MESH,
        scratch_types=[
            pltpu.VMEM((EC,), jnp.int32),
            pltpu.VMEM((EC, W), F32),
            pltpu.VMEM((EC, W), F32),
            pltpu.SemaphoreType.DMA,
            pltpu.SemaphoreType.DMA,
        ],
    )
    o = fin(q[0], q[1], idxp)
    return o[:nidx, 1:1 + od]


# speedup vs baseline: 1.2981x; 1.2981x over previous
import functools

import jax
import jax.numpy as jnp
import numpy as np
from jax import lax
from jax.experimental import pallas as pl
from jax.experimental.pallas import tpu as pltpu
from jax.experimental.pallas import tpu_sc as plsc

F32 = jnp.float32
NC, NS, LANES = 2, 16, 16
NW = NC * NS
EC = 80
W = 128

_MESH = plsc.VectorSubcoreMesh(
    core_axis_name="c", subcore_axis_name="s", num_cores=NC, num_subcores=NS
)

_GDN = lax.GatherDimensionNumbers(
    offset_dims=(), collapsed_slice_dims=(0,), start_index_map=(0,)
)


def _vgather(v, idx):
    return lax.gather(v, idx[:, None], _GDN, (1,),
                      mode=lax.GatherScatterMode.PROMISE_IN_BOUNDS)


def _prep1_body(x_ref, w_ref, msa_ref, mtb_ref, mhb_ref, a_ref, b_ref):
    h = jnp.dot(x_ref[...], w_ref[...], preferred_element_type=F32)
    lr = jnp.where(h > 0, h, 0.2 * h)
    a_ref[...] = jnp.dot(lr, msa_ref[...], preferred_element_type=F32)
    b_ref[...] = (jnp.dot(lr, mtb_ref[...], preferred_element_type=F32)
                  + jnp.dot(h, mhb_ref[...], preferred_element_type=F32))


def _prep2_body(p0_ref, p1_ref, md_ref, mn_ref, w2_ref, msa2_ref, mtb2_ref,
                mhb2_ref, a2_ref, b2_ref):
    acc = p0_ref[...] + p1_ref[...]
    den = jnp.dot(acc, md_ref[...], preferred_element_type=F32)
    num = jnp.dot(acc, mn_ref[...], preferred_element_type=F32)
    x2 = jnp.maximum(num / (den + 1e-9), 0.0)
    h2 = jnp.dot(x2, w2_ref[...], preferred_element_type=F32)
    lr2 = jnp.where(h2 > 0, h2, 0.2 * h2)
    a2_ref[...] = jnp.dot(lr2, msa2_ref[...], preferred_element_type=F32)
    b2_ref[...] = (jnp.dot(lr2, mtb2_ref[...], preferred_element_type=F32)
                   + jnp.dot(h2, mhb2_ref[...], preferred_element_type=F32))


ECB = 128


def _edge_body(src_hbm, tgt_hbm, a_hbm, b_hbm, out_hbm,
               acc_sh, srcv, tgtv, sbuf, tbuf, contrib, sem_a, sem_b,
               *, layer):
    core = lax.axis_index("c")
    sub = lax.axis_index("s")
    wid = sub * NC + core
    n_pad = acc_sh.shape[0]
    nper = src_hbm.shape[0] // NW
    nblocks = nper // ECB
    rps = n_pad // NS
    chunks = []
    r = 0
    while r < rps:
        c = min(ECB, rps - r)
        chunks.append((r, c))
        r += c
    zeros16 = jnp.zeros((LANES,), F32)
    ii = lax.iota(jnp.int32, LANES)
    rs = sub * rps

    def _zrow(i, c):
        for o in range(0, W, LANES):
            contrib[i, pl.ds(o, LANES)] = zeros16
        return c
    lax.fori_loop(0, ECB, _zrow, 0)
    for r0c, cc in chunks:
        pltpu.sync_copy(contrib.at[pl.ds(0, cc)], acc_sh.at[pl.ds(rs + r0c, cc)])
    plsc.subcore_barrier()

    bidx = [(ii >> 3) + 2 * c for c in range(4)]
    zidx = ii * 0

    def _block(b, c):
        base = wid * nper + b * ECB
        pltpu.sync_copy(src_hbm.at[pl.ds(base, ECB)], srcv)
        pltpu.sync_copy(tgt_hbm.at[pl.ds(base, ECB)], tgtv)
        cpa = pltpu.async_copy(a_hbm.at[srcv], sbuf, sem_a)
        cpb = pltpu.async_copy(b_hbm.at[tgtv], tbuf, sem_b)
        cpa.wait()
        cpb.wait()

        if layer == 1:
            def _edge(i, c2):
                sv = sbuf[i, pl.ds(0, LANES)]
                tv = tbuf[i, pl.ds(0, LANES)]
                ev = jnp.exp(jnp.clip(sv + tv, -2.0, 2.0))
                contrib[i, pl.ds(0, LANES)] = ev
                for v in range(4):
                    hc = tbuf[i, pl.ds(8 + LANES * v, LANES)]
                    eb = _vgather(ev, bidx[v])
                    contrib[i, pl.ds(8 + LANES * v, LANES)] = eb * hc
                return c2
        else:
            def _edge(i, c2):
                sv = sbuf[i, pl.ds(0, LANES)]
                tv = tbuf[i, pl.ds(0, LANES)]
                sm = sv + tv
                ev = jnp.exp(jnp.clip(sm, -2.0, 2.0))
                e2 = _vgather(ev, zidx)
                contrib[i, pl.ds(0, LANES)] = e2 * jnp.where(ii == 0, 1.0, sm)
                return c2
        lax.fori_loop(0, ECB, _edge, 0)
        pltpu.sync_copy(contrib, acc_sh.at[srcv], add=True)
        return c
    lax.fori_loop(0, nblocks, _block, 0)

    plsc.subcore_barrier()
    for r0c, cc in chunks:
        pltpu.sync_copy(acc_sh.at[pl.ds(rs + r0c, cc)], contrib.at[pl.ds(0, cc)])
        pltpu.sync_copy(contrib.at[pl.ds(0, cc)], out_hbm.at[core, pl.ds(rs + r0c, cc)])


def _fin_body(q0_hbm, q1_hbm, idx_hbm, out_hbm, idxv, r0, r1, sem0, sem1):
    core = lax.axis_index("c")
    sub = lax.axis_index("s")
    wid = sub * NC + core
    per = idx_hbm.shape[0] // NW
    nb = per // EC
    ii = lax.iota(jnp.int32, LANES)
    zidx = ii * 0

    def _block(b, c):
        base = wid * per + b * EC
        pltpu.sync_copy(idx_hbm.at[pl.ds(base, EC)], idxv)
        cp0 = pltpu.async_copy(q0_hbm.at[idxv], r0, sem0)
        cp1 = pltpu.async_copy(q1_hbm.at[idxv], r1, sem1)
        cp0.wait()
        cp1.wait()

        def _row(i, c2):
            srow = r0[i, pl.ds(0, LANES)] + r1[i, pl.ds(0, LANES)]
            den = _vgather(srow, zidx)
            r0[i, pl.ds(0, LANES)] = jnp.maximum(srow / (den + 1e-9), 0.0)
            return c2
        lax.fori_loop(0, EC, _row, 0)
        pltpu.sync_copy(r0, out_hbm.at[pl.ds(base, EC)])
        return c
    lax.fori_loop(0, nb, _block, 0)


def _make_edge_call(n_pad, layer):
    body = functools.partial(_edge_body, layer=layer)
    return pl.kernel(
        body,
        out_type=jax.ShapeDtypeStruct((NC, n_pad, W), F32),
        mesh=_MESH,
        scratch_types=[
            pltpu.VMEM_SHARED((n_pad, W), F32),
            pltpu.VMEM((ECB,), jnp.int32),
            pltpu.VMEM((ECB,), jnp.int32),
            pltpu.VMEM((ECB, W), F32),
            pltpu.VMEM((ECB, W), F32),
            pltpu.VMEM((ECB, W), F32),
            pltpu.SemaphoreType.DMA,
            pltpu.SemaphoreType.DMA,
        ],
    )


def kernel(node_states, edges, indices, W1, a1, W2, a2):
    n, d = node_states.shape
    h1, _, u1 = W1.shape
    hu = h1 * u1
    od = W2.shape[2]
    nidx = indices.shape[0]

    w1f = jnp.transpose(W1, (1, 0, 2)).reshape(d, hu)
    rows = np.arange(hu)
    heads = rows // u1
    msa = jnp.zeros((hu, W), F32).at[rows, heads].set(a1[:, :u1, 0].reshape(hu))
    mtb = jnp.zeros((hu, W), F32).at[rows, heads].set(a1[:, u1:, 0].reshape(hu))
    mhb = np.zeros((hu, W), np.float32)
    mhb[rows, rows + 8] = 1.0
    mhb = jnp.asarray(mhb)
    md = np.zeros((W, hu), np.float32)
    md[heads, rows] = 1.0
    md = jnp.asarray(md)
    mn = np.zeros((W, hu), np.float32)
    mn[rows + 8, rows] = 1.0
    mn = jnp.asarray(mn)
    w2p = jnp.concatenate([W2[0], jnp.zeros((hu, 8 - od), F32)], axis=1)
    j7 = np.arange(od)
    msa2 = jnp.zeros((8, W), F32).at[j7, 0].set(a2[0, :od, 0])
    mtb2 = jnp.zeros((8, W), F32).at[j7, 0].set(a2[0, od:, 0])
    mhb2 = np.zeros((8, W), np.float32)
    mhb2[j7, j7 + 1] = 1.0
    mhb2 = jnp.asarray(mhb2)

    e_num = edges.shape[0]
    per_tile = ((e_num + NW * ECB - 1) // (NW * ECB)) * ECB
    e_pad = NW * per_tile
    src = jnp.concatenate(
        [edges[:, 0], jnp.full((e_pad - e_num,), n, jnp.int32)])
    tgt = jnp.concatenate(
        [edges[:, 1], jnp.zeros((e_pad - e_num,), jnp.int32)])

    npd = ((n + 1 + W - 1) // W) * W
    xp = jnp.concatenate([node_states, jnp.zeros((npd - n, d), F32)], axis=0)
    blk = 1264
    grid = (npd // blk,)
    full = lambda i: (0, 0)
    rowb = lambda i: (i, 0)
    prep1 = pl.pallas_call(
        _prep1_body,
        grid=grid,
        in_specs=[
            pl.BlockSpec((blk, d), rowb),
            pl.BlockSpec((d, hu), full),
            pl.BlockSpec((hu, W), full),
            pl.BlockSpec((hu, W), full),
            pl.BlockSpec((hu, W), full),
        ],
        out_specs=[pl.BlockSpec((blk, W), rowb), pl.BlockSpec((blk, W), rowb)],
        out_shape=[jax.ShapeDtypeStruct((npd, W), F32),
                   jax.ShapeDtypeStruct((npd, W), F32)],
    )
    tab_a1, tab_b1 = prep1(xp, w1f, msa, mtb, mhb)

    edge1 = _make_edge_call(npd, 1)
    p1 = edge1(src, tgt, tab_a1, tab_b1)

    prep2 = pl.pallas_call(
        _prep2_body,
        grid=grid,
        in_specs=[
            pl.BlockSpec((blk, W), rowb),
            pl.BlockSpec((blk, W), rowb),
            pl.BlockSpec((W, hu), full),
            pl.BlockSpec((W, hu), full),
            pl.BlockSpec((hu, 8), full),
            pl.BlockSpec((8, W), full),
            pl.BlockSpec((8, W), full),
            pl.BlockSpec((8, W), full),
        ],
        out_specs=[pl.BlockSpec((blk, W), rowb), pl.BlockSpec((blk, W), rowb)],
        out_shape=[jax.ShapeDtypeStruct((npd, W), F32),
                   jax.ShapeDtypeStruct((npd, W), F32)],
    )
    tab_a2, tab_b2 = prep2(p1[0], p1[1], md, mn, w2p, msa2, mtb2, mhb2)

    edge2 = _make_edge_call(npd, 2)
    q = edge2(src, tgt, tab_a2, tab_b2)

    npad = ((nidx + NW * EC - 1) // (NW * EC)) * (NW * EC)
    idxp = jnp.concatenate([indices, jnp.zeros((npad - nidx,), jnp.int32)])
    fin = pl.kernel(
        _fin_body,
        out_type=jax.ShapeDtypeStruct((npad, W), F32),
        mesh=_MESH,
        scratch_types=[
            pltpu.VMEM((EC,), jnp.int32),
            pltpu.VMEM((EC, W), F32),
            pltpu.VMEM((EC, W), F32),
            pltpu.SemaphoreType.DMA,
            pltpu.SemaphoreType.DMA,
        ],
    )
    o = fin(q[0], q[1], idxp)
    return o[:nidx, 1:1 + od]


# speedup vs baseline: 1.5511x; 1.1950x over previous
import functools

import jax
import jax.numpy as jnp
import numpy as np
from jax import lax
from jax.experimental import pallas as pl
from jax.experimental.pallas import tpu as pltpu
from jax.experimental.pallas import tpu_sc as plsc

F32 = jnp.float32
NC, NS, LANES = 2, 16, 16
NW = NC * NS
EC = 80
W = 128

_MESH = plsc.VectorSubcoreMesh(
    core_axis_name="c", subcore_axis_name="s", num_cores=NC, num_subcores=NS
)

_GDN = lax.GatherDimensionNumbers(
    offset_dims=(), collapsed_slice_dims=(0,), start_index_map=(0,)
)


def _vgather(v, idx):
    return lax.gather(v, idx[:, None], _GDN, (1,),
                      mode=lax.GatherScatterMode.PROMISE_IN_BOUNDS)


def _prep1_body(x_ref, w_ref, msa_ref, mtb_ref, mhb_ref, a_ref, b_ref):
    h = jnp.dot(x_ref[...], w_ref[...], preferred_element_type=F32)
    lr = jnp.where(h > 0, h, 0.2 * h)
    a_ref[...] = jnp.dot(lr, msa_ref[...], preferred_element_type=F32)
    b_ref[...] = (jnp.dot(lr, mtb_ref[...], preferred_element_type=F32)
                  + jnp.dot(h, mhb_ref[...], preferred_element_type=F32))


def _prep2_body(p0_ref, p1_ref, md_ref, mn_ref, w2_ref, msa2_ref, mtb2_ref,
                mhb2_ref, a2_ref, b2_ref):
    acc = p0_ref[...] + p1_ref[...]
    den = jnp.dot(acc, md_ref[...], preferred_element_type=F32)
    num = jnp.dot(acc, mn_ref[...], preferred_element_type=F32)
    x2 = jnp.maximum(num / (den + 1e-9), 0.0)
    h2 = jnp.dot(x2, w2_ref[...], preferred_element_type=F32)
    lr2 = jnp.where(h2 > 0, h2, 0.2 * h2)
    a2_ref[...] = jnp.dot(lr2, msa2_ref[...], preferred_element_type=F32)
    b2_ref[...] = (jnp.dot(lr2, mtb2_ref[...], preferred_element_type=F32)
                   + jnp.dot(h2, mhb2_ref[...], preferred_element_type=F32))


def _edge_body(src_hbm, tgt_hbm, a_hbm, b_hbm, out_hbm,
               acc_sh, srcv, tgtv, sbuf, tbuf, contrib, xbuf, sem_a, sem_b,
               *, layer):
    core = lax.axis_index("c")
    sub = lax.axis_index("s")
    wid = sub * NC + core
    n_pad = acc_sh.shape[0]
    wa = acc_sh.shape[1]
    nper = src_hbm.shape[0] // NW
    nblocks = nper // EC
    rps = n_pad // NS
    nchunks = rps // EC
    zeros16 = jnp.zeros((LANES,), F32)
    ii = lax.iota(jnp.int32, LANES)
    rs = sub * rps

    zo = list(range(0, wa - LANES + 1, LANES))
    if zo[-1] != wa - LANES:
        zo.append(wa - LANES)

    def _zrow(i, c):
        for o in range(0, W, LANES):
            xbuf[i, pl.ds(o, LANES)] = zeros16
        for o in zo:
            contrib[i, pl.ds(o, LANES)] = zeros16
        return c
    lax.fori_loop(0, EC, _zrow, 0)
    for k in range(nchunks):
        pltpu.sync_copy(contrib, acc_sh.at[pl.ds(rs + k * EC, EC)])
    plsc.subcore_barrier()

    bidx = [(ii >> 3) + 2 * c for c in range(4)]
    zidx = ii * 0

    def _block(b, c):
        base = wid * nper + b * EC
        pltpu.sync_copy(src_hbm.at[pl.ds(base, EC)], srcv)
        pltpu.sync_copy(tgt_hbm.at[pl.ds(base, EC)], tgtv)
        cpa = pltpu.async_copy(a_hbm.at[srcv], sbuf, sem_a)
        cpb = pltpu.async_copy(b_hbm.at[tgtv], tbuf, sem_b)
        cpa.wait()
        cpb.wait()

        if layer == 1:
            @plsc.parallel_loop(0, EC, unroll=4)
            def _edge(i):
                sv = sbuf[i, pl.ds(0, LANES)]
                tv = tbuf[i, pl.ds(0, LANES)]
                ev = jnp.exp(jnp.clip(sv + tv, -2.0, 2.0))
                contrib[i, pl.ds(0, LANES)] = ev
                for v in range(4):
                    hc = tbuf[i, pl.ds(8 + LANES * v, LANES)]
                    eb = _vgather(ev, bidx[v])
                    contrib[i, pl.ds(8 + LANES * v, LANES)] = eb * hc
        else:
            @plsc.parallel_loop(0, EC, unroll=4)
            def _edge(i):
                sv = sbuf[i, pl.ds(0, LANES)]
                tv = tbuf[i, pl.ds(0, LANES)]
                sm = sv + tv
                ev = jnp.exp(jnp.clip(sm, -2.0, 2.0))
                e2 = _vgather(ev, zidx)
                contrib[i, pl.ds(0, LANES)] = e2 * jnp.where(ii == 0, 1.0, sm)
        pltpu.sync_copy(contrib, acc_sh.at[srcv], add=True)
        return c
    lax.fori_loop(0, nblocks, _block, 0)

    plsc.subcore_barrier()
    for k in range(nchunks):
        pltpu.sync_copy(acc_sh.at[pl.ds(rs + k * EC, EC)], contrib)

        def _xrow(i, c):
            for o in zo:
                xbuf[i, pl.ds(o, LANES)] = contrib[i, pl.ds(o, LANES)]
            return c
        lax.fori_loop(0, EC, _xrow, 0)
        pltpu.sync_copy(xbuf, out_hbm.at[core, pl.ds(rs + k * EC, EC)])


def _fin_body(q0_hbm, q1_hbm, idx_hbm, out_hbm, idxv, r0, r1, sem0, sem1):
    core = lax.axis_index("c")
    sub = lax.axis_index("s")
    wid = sub * NC + core
    per = idx_hbm.shape[0] // NW
    nb = per // EC
    ii = lax.iota(jnp.int32, LANES)
    zidx = ii * 0

    def _block(b, c):
        base = wid * per + b * EC
        pltpu.sync_copy(idx_hbm.at[pl.ds(base, EC)], idxv)
        cp0 = pltpu.async_copy(q0_hbm.at[idxv], r0, sem0)
        cp1 = pltpu.async_copy(q1_hbm.at[idxv], r1, sem1)
        cp0.wait()
        cp1.wait()

        def _row(i, c2):
            srow = r0[i, pl.ds(0, LANES)] + r1[i, pl.ds(0, LANES)]
            den = _vgather(srow, zidx)
            r0[i, pl.ds(0, LANES)] = jnp.maximum(srow / (den + 1e-9), 0.0)
            return c2
        lax.fori_loop(0, EC, _row, 0)
        pltpu.sync_copy(r0, out_hbm.at[pl.ds(base, EC)])
        return c
    lax.fori_loop(0, nb, _block, 0)


def _make_edge_call(n_pad, layer):
    body = functools.partial(_edge_body, layer=layer)
    wa = W
    return pl.kernel(
        body,
        out_type=jax.ShapeDtypeStruct((NC, n_pad, W), F32),
        mesh=_MESH,
        scratch_types=[
            pltpu.VMEM_SHARED((n_pad, wa), F32),
            pltpu.VMEM((EC,), jnp.int32),
            pltpu.VMEM((EC,), jnp.int32),
            pltpu.VMEM((EC, W), F32),
            pltpu.VMEM((EC, W), F32),
            pltpu.VMEM((EC, wa), F32),
            pltpu.VMEM((EC, W), F32),
            pltpu.SemaphoreType.DMA,
            pltpu.SemaphoreType.DMA,
        ],
    )


def kernel(node_states, edges, indices, W1, a1, W2, a2):
    n, d = node_states.shape
    h1, _, u1 = W1.shape
    hu = h1 * u1
    od = W2.shape[2]
    nidx = indices.shape[0]

    w1f = jnp.transpose(W1, (1, 0, 2)).reshape(d, hu)
    rows = np.arange(hu)
    heads = rows // u1
    msa = jnp.zeros((hu, W), F32).at[rows, heads].set(a1[:, :u1, 0].reshape(hu))
    mtb = jnp.zeros((hu, W), F32).at[rows, heads].set(a1[:, u1:, 0].reshape(hu))
    mhb = np.zeros((hu, W), np.float32)
    mhb[rows, rows + 8] = 1.0
    mhb = jnp.asarray(mhb)
    md = np.zeros((W, hu), np.float32)
    md[heads, rows] = 1.0
    md = jnp.asarray(md)
    mn = np.zeros((W, hu), np.float32)
    mn[rows + 8, rows] = 1.0
    mn = jnp.asarray(mn)
    w2p = jnp.concatenate([W2[0], jnp.zeros((hu, 8 - od), F32)], axis=1)
    j7 = np.arange(od)
    msa2 = jnp.zeros((8, W), F32).at[j7, 0].set(a2[0, :od, 0])
    mtb2 = jnp.zeros((8, W), F32).at[j7, 0].set(a2[0, od:, 0])
    mhb2 = np.zeros((8, W), np.float32)
    mhb2[j7, j7 + 1] = 1.0
    mhb2 = jnp.asarray(mhb2)

    src = edges[:, 0]
    tgt = edges[:, 1]

    npd = ((n + NS * EC - 1) // (NS * EC)) * (NS * EC)
    xp = jnp.concatenate([node_states, jnp.zeros((npd - n, d), F32)], axis=0)
    blk = 2048
    grid = (npd // blk,)
    full = lambda i: (0, 0)
    rowb = lambda i: (i, 0)
    prep1 = pl.pallas_call(
        _prep1_body,
        grid=grid,
        in_specs=[
            pl.BlockSpec((blk, d), rowb),
            pl.BlockSpec((d, hu), full),
            pl.BlockSpec((hu, W), full),
            pl.BlockSpec((hu, W), full),
            pl.BlockSpec((hu, W), full),
        ],
        out_specs=[pl.BlockSpec((blk, W), rowb), pl.BlockSpec((blk, W), rowb)],
        out_shape=[jax.ShapeDtypeStruct((npd, W), F32),
                   jax.ShapeDtypeStruct((npd, W), F32)],
    )
    tab_a1, tab_b1 = prep1(xp, w1f, msa, mtb, mhb)

    edge1 = _make_edge_call(npd, 1)
    p1 = edge1(src, tgt, tab_a1, tab_b1)

    prep2 = pl.pallas_call(
        _prep2_body,
        grid=grid,
        in_specs=[
            pl.BlockSpec((blk, W), rowb),
            pl.BlockSpec((blk, W), rowb),
            pl.BlockSpec((W, hu), full),
            pl.BlockSpec((W, hu), full),
            pl.BlockSpec((hu, 8), full),
            pl.BlockSpec((8, W), full),
            pl.BlockSpec((8, W), full),
            pl.BlockSpec((8, W), full),
        ],
        out_specs=[pl.BlockSpec((blk, W), rowb), pl.BlockSpec((blk, W), rowb)],
        out_shape=[jax.ShapeDtypeStruct((npd, W), F32),
                   jax.ShapeDtypeStruct((npd, W), F32)],
    )
    tab_a2, tab_b2 = prep2(p1[0], p1[1], md, mn, w2p, msa2, mtb2, mhb2)

    edge2 = _make_edge_call(npd, 2)
    q = edge2(src, tgt, tab_a2, tab_b2)

    npad = ((nidx + NW * EC - 1) // (NW * EC)) * (NW * EC)
    idxp = jnp.concatenate([indices, jnp.zeros((npad - nidx,), jnp.int32)])
    fin = pl.kernel(
        _fin_body,
        out_type=jax.ShapeDtypeStruct((npad, W), F32),
        mesh=_MESH,
        scratch_types=[
            pltpu.VMEM((EC,), jnp.int32),
            pltpu.VMEM((EC, W), F32),
            pltpu.VMEM((EC, W), F32),
            pltpu.SemaphoreType.DMA,
            pltpu.SemaphoreType.DMA,
        ],
    )
    o = fin(q[0], q[1], idxp)
    return o[:nidx, 1:1 + od]
